# baseline (speedup 1.0000x reference)
import functools

import jax
import jax.numpy as jnp
from jax import lax
from jax.experimental import pallas as pl
from jax.experimental.pallas import tpu as pltpu
from jax.experimental.pallas import tpu_sc as plsc

N = 8192
APP_DIM = 48
KB = 1024
M = 64
GW = 64
SW = 32


def _sigmoid(x):
    return 1.0 / (1.0 + jnp.exp(-x))


def _diffuse_kernel(xyz_ref, vd_ref, n_ref, app_ref, noise_ref,
                    W1_ref, b1_ref, W2_ref, b2_ref,
                    diffuse_ref, r1_ref, table_ref, bg_ref):
    x3 = xyz_ref[...][:, 0:3]
    v = vd_ref[...]
    a = app_ref[...]
    W1 = W1_ref[...]
    h = (jnp.dot(x3, W1[0:3, :], preferred_element_type=jnp.float32)
         + jnp.dot(v, W1[3:6, :], preferred_element_type=jnp.float32)
         + jnp.dot(a, W1[6:54, :], preferred_element_type=jnp.float32)
         + b1_ref[...])
    h = jnp.maximum(h, 0.0)
    o = jnp.dot(h, W2_ref[...], preferred_element_type=jnp.float32) + b2_ref[...]
    diffuse = _sigmoid(o[:, 0:3])
    diffuse_ref[...] = diffuse
    r1 = _sigmoid(o[:, 6:7]) * 0.99 + 0.01
    r1_ref[...] = r1
    noise_app = a + 0.1 * noise_ref[...]
    blk = x3.shape[0]
    table_ref[...] = jnp.concatenate(
        [n_ref[...], v, x3, r1, noise_app, diffuse,
         jnp.zeros((blk, GW - 61), dtype=jnp.float32)], axis=1)
    bg_ref[...] = jnp.concatenate(
        [diffuse, jnp.zeros((blk, SW - 3), dtype=jnp.float32)], axis=1)


def _run_diffuse(xyzs, viewdirs, normals, app, noise, Wd1, bd1, Wd2, bd2):
    blk = 2048
    grid = (N // blk,)
    row = lambda i: (i, 0)
    full = lambda i: (0, 0)
    return pl.pallas_call(
        _diffuse_kernel,
        grid=grid,
        in_specs=[
            pl.BlockSpec((blk, 4), row),
            pl.BlockSpec((blk, 3), row),
            pl.BlockSpec((blk, 3), row),
            pl.BlockSpec((blk, APP_DIM), row),
            pl.BlockSpec((blk, APP_DIM), row),
            pl.BlockSpec((54, 64), full),
            pl.BlockSpec((1, 64), full),
            pl.BlockSpec((64, 8), full),
            pl.BlockSpec((1, 8), full),
        ],
        out_specs=[
            pl.BlockSpec((blk, 3), row),
            pl.BlockSpec((blk, 1), row),
            pl.BlockSpec((blk, GW), row),
            pl.BlockSpec((blk, SW), row),
        ],
        out_shape=[
            jax.ShapeDtypeStruct((N, 3), jnp.float32),
            jax.ShapeDtypeStruct((N, 1), jnp.float32),
            jax.ShapeDtypeStruct((N, GW), jnp.float32),
            jax.ShapeDtypeStruct((N, SW), jnp.float32),
        ],
    )(xyzs, viewdirs, normals, app, noise, Wd1, bd1, Wd2, bd2)


_NW = 32
_ROWS_PER_W = KB // _NW


@functools.lru_cache(maxsize=None)
def _sc_mesh():
    return plsc.VectorSubcoreMesh(core_axis_name="c", subcore_axis_name="s")


@functools.lru_cache(maxsize=None)
def _make_sc_gather():
    @functools.partial(
        pl.kernel,
        out_type=jax.ShapeDtypeStruct((KB, GW), jnp.float32),
        mesh=_sc_mesh(),
        compiler_params=pltpu.CompilerParams(use_tc_tiling_on_sc=False),
        scratch_types=[
            pltpu.VMEM((_ROWS_PER_W,), jnp.int32),
            pltpu.VMEM((_ROWS_PER_W, GW), jnp.float32),
            pltpu.SemaphoreType.DMA,
        ],
    )
    def _sc_gather(table_hbm, idx_hbm, out_hbm, idx_v, rows_v, sem):
        wid = lax.axis_index("s") * 2 + lax.axis_index("c")
        base = wid * _ROWS_PER_W
        pltpu.sync_copy(idx_hbm.at[pl.ds(base, _ROWS_PER_W)], idx_v)
        pltpu.async_copy(table_hbm.at[idx_v], rows_v, sem).wait()
        pltpu.sync_copy(rows_v, out_hbm.at[pl.ds(base, _ROWS_PER_W)])

    return _sc_gather


def _ray_kernel(gT_ref, u1T_ref, u2T_ref, maskT_ref, WrT_ref, b1T_ref,
                WcT_ref, W2T_ref, b2T_ref, Wenv_ref, outT_ref):
    gT = gT_ref[...]
    nx, ny, nz = gT[0:1, :], gT[1:2, :], gT[2:3, :]
    vx, vy, vz = -gT[3:4, :], -gT[4:5, :], -gT[5:6, :]
    ea = gT[9:10, :]

    cond = jnp.abs(nz) < 0.9
    tux = jnp.where(cond, -ny, 0.0)
    tuy = jnp.where(cond, nx, -nz)
    tuz = jnp.where(cond, 0.0, ny)
    tn = jnp.sqrt(tux * tux + tuy * tuy + tuz * tuz) + 1e-8
    tx, ty, tz = tux / tn, tuy / tn, tuz / tn
    bx = ny * tz - nz * ty
    by = nz * tx - nx * tz
    bz = nx * ty - ny * tx

    alpha = ea * ea
    a2 = alpha * alpha
    u1 = u1T_ref[...]
    u2 = u2T_ref[...]
    phi = (2.0 * jnp.pi) * u1
    ct = jnp.sqrt((1.0 - u2) / (1.0 + (a2 - 1.0) * u2))
    st = jnp.sqrt(jnp.clip(1.0 - ct * ct, 0.0, 1.0))
    h0 = st * jnp.cos(phi)
    h1 = st * jnp.sin(phi)
    h2 = ct

    hwx = tx * h0 + bx * h1 + nx * h2
    hwy = ty * h0 + by * h1 + ny * h2
    hwz = tz * h0 + bz * h1 + nz * h2
    hn = jnp.sqrt(hwx * hwx + hwy * hwy + hwz * hwz) + 1e-8
    hwx, hwy, hwz = hwx / hn, hwy / hn, hwz / hn

    dvh = vx * hwx + vy * hwy + vz * hwz
    lxu = 2.0 * dvh * hwx - vx
    lyu = 2.0 * dvh * hwy - vy
    lzu = 2.0 * dvh * hwz - vz
    ln = jnp.sqrt(lxu * lxu + lyu * lyu + lzu * lzu) + 1e-8
    lx, ly, lz = lxu / ln, lyu / ln, lzu / ln

    hxu = (vx + lx) * 0.5
    hyu = (vy + ly) * 0.5
    hzu = (vz + lz) * 0.5
    hn2 = jnp.sqrt(hxu * hxu + hyu * hyu + hzu * hzu) + 1e-8
    hx, hy, hz = hxu / hn2, hyu / hn2, hzu / hn2

    noh = jnp.clip(nx * hx + ny * hy + nz * hz, 1e-6, 1.0)
    voh = jnp.clip(jnp.abs(hx * vx + hy * vy + hz * vz), 1e-6, 1.0)
    mip = jnp.log(alpha / (4.0 * noh * voh + 1e-8) + 1e-8)

    wenv = Wenv_ref[...]
    il = []
    for c in range(3):
        acc = (lx * wenv[0:1, c:c + 1] + ly * wenv[1:2, c:c + 1]
               + lz * wenv[2:3, c:c + 1] + mip * wenv[3:4, c:c + 1])
        il.append(_sigmoid(acc))

    dv0 = tx * lx + ty * ly + tz * lz
    dv1 = bx * lx + by * ly + bz * lz
    dv2 = nx * lx + ny * ly + nz * lz
    hv0 = tx * hx + ty * hy + tz * hz
    hv1 = bx * hx + by * hy + bz * hz
    hv2 = nx * hx + ny * hy + nz * hz

    row_inT = jnp.concatenate(
        [-gT[3:6, :], gT[0:3, :], gT[10:58, :], gT[9:10, :]], axis=0)
    row_acc = (jnp.dot(WrT_ref[...], row_inT, preferred_element_type=jnp.float32)
               + b1T_ref[...])

    WcT = WcT_ref[...]
    W2T = W2T_ref[...]
    b2T = b2T_ref[...]
    maskT = maskT_ref[...]
    cnt = jnp.sum(maskT, axis=0, keepdims=True) + 1e-8

    tint_acc = jnp.zeros((3, KB), dtype=jnp.float32)
    brdf_acc = jnp.zeros((3, KB), dtype=jnp.float32)
    for m in range(M):
        cf = jnp.concatenate(
            [lx[m:m + 1, :], ly[m:m + 1, :], lz[m:m + 1, :],
             hv0[m:m + 1, :], hv1[m:m + 1, :], hv2[m:m + 1, :],
             dv0[m:m + 1, :], dv1[m:m + 1, :], dv2[m:m + 1, :]], axis=0)
        bh = jnp.maximum(
            row_acc + jnp.dot(WcT, cf, preferred_element_type=jnp.float32), 0.0)
        bw = _sigmoid(jnp.dot(W2T, bh, preferred_element_type=jnp.float32) + b2T)
        ilm = jnp.concatenate(
            [il[0][m:m + 1, :], il[1][m:m + 1, :], il[2][m:m + 1, :]], axis=0)
        mk = maskT[m:m + 1, :]
        tint_acc = tint_acc + mk * ilm * bw
        brdf_acc = brdf_acc + mk * bw

    spec_rows = [jnp.sum(maskT * il[c], axis=0, keepdims=True) for c in range(3)]
    pad5 = jnp.zeros((5, KB), dtype=jnp.float32)
    packed = jnp.concatenate(
        [tint_acc / cnt + gT[58:61, :],
         pad5,
         brdf_acc / cnt,
         pad5,
         jnp.concatenate(spec_rows, axis=0) / cnt,
         jnp.zeros((SW - 19, KB), dtype=jnp.float32)], axis=0)
    outT_ref[...] = packed.T


def _run_rays(gbT, u1T, u2T, maskT, WrT, b1T, WcT, W2T, b2T, Wenv):
    full = lambda: (0, 0)
    return pl.pallas_call(
        _ray_kernel,
        in_specs=[
            pl.BlockSpec((GW, KB), full),
            pl.BlockSpec((M, KB), full),
            pl.BlockSpec((M, KB), full),
            pl.BlockSpec((M, KB), full),
            pl.BlockSpec((64, 55), full),
            pl.BlockSpec((64, 1), full),
            pl.BlockSpec((64, 9), full),
            pl.BlockSpec((3, 64), full),
            pl.BlockSpec((3, 1), full),
            pl.BlockSpec((4, 3), full),
        ],
        out_specs=pl.BlockSpec((KB, SW), full),
        out_shape=jax.ShapeDtypeStruct((KB, SW), jnp.float32),
    )(gbT, u1T, u2T, maskT, WrT, b1T, WcT, W2T, b2T, Wenv)


_ZROWS = N // 16
_SROWS = KB // 16


@functools.lru_cache(maxsize=None)
def _make_sc_scatter():
    @functools.partial(
        pl.kernel,
        out_type=[
            jax.ShapeDtypeStruct((N, SW), jnp.float32),
            jax.ShapeDtypeStruct((N, 3), jnp.float32),
            jax.ShapeDtypeStruct((N, 3), jnp.float32),
            jax.ShapeDtypeStruct((N, 3), jnp.float32),
        ],
        mesh=_sc_mesh(),
        compiler_params=pltpu.CompilerParams(use_tc_tiling_on_sc=False),
        scratch_types=[
            pltpu.VMEM((_ZROWS, SW), jnp.float32),
            pltpu.VMEM((_SROWS,), jnp.int32),
            pltpu.VMEM((_SROWS, SW), jnp.float32),
            pltpu.VMEM((_ZROWS, 3), jnp.float32),
            pltpu.SemaphoreType.DMA,
        ],
    )
    def _sc_scatter(vals_hbm, idx_hbm, bg_hbm, out_hbm, rgb_hbm, brdf_hbm,
                    spec_hbm, zbuf, idx_v, vals_v, cbuf, sem):
        cid = lax.axis_index("c")
        sid = lax.axis_index("s")

        @pl.when(cid == 0)
        def _():
            s = pl.ds(sid * _ZROWS, _ZROWS)
            pltpu.sync_copy(bg_hbm.at[s], zbuf)
            pltpu.sync_copy(zbuf, out_hbm.at[s])
            plsc.subcore_barrier()
            pltpu.sync_copy(idx_hbm.at[pl.ds(sid * _SROWS, _SROWS)], idx_v)
            pltpu.sync_copy(vals_hbm.at[pl.ds(sid * _SROWS, _SROWS)], vals_v)
            pltpu.async_copy(vals_v, out_hbm.at[idx_v], sem).wait()
            plsc.subcore_barrier()
            for dst, c0 in ((rgb_hbm, 0), (brdf_hbm, 8), (spec_hbm, 16)):
                pltpu.sync_copy(out_hbm.at[s, pl.ds(c0, 3)], cbuf)
                pltpu.sync_copy(cbuf, dst.at[s])

    return _sc_scatter


def kernel(xyzs, app_features, viewdirs, normals, weights, feat_noise, u1, u2,
           Wd1, bd1, Wd2, bd2, Wb1, bb1, Wb2, bb2, Wenv,
           app_mask, bounce_idx, ray_rows, ray_cols, ray_mask):
    diffuse, r1, table, bg = _run_diffuse(
        xyzs, viewdirs, normals, app_features, feat_noise,
        Wd1, bd1.reshape(1, 64), Wd2, bd2.reshape(1, 8))

    gb = _make_sc_gather()(table, bounce_idx)

    maskT = ray_mask.astype(jnp.float32).T
    WrT = jnp.concatenate([Wb1[0:3], Wb1[6:9], Wb1[15:63], Wb1[63:64]], axis=0).T
    WcT = Wb1[jnp.array([3, 4, 5, 9, 10, 11, 12, 13, 14])].T
    vals = _run_rays(gb.T, u1.T, u2.T, maskT, WrT, bb1.reshape(64, 1),
                     WcT, Wb2.T, bb2.reshape(3, 1), Wenv)

    _, rgb, brdf_rgb, spec = _make_sc_scatter()(vals, bounce_idx, bg)

    return (rgb, diffuse, r1, brdf_rgb, spec)


# speedup vs baseline: 1.0588x; 1.0588x over previous
import functools

import jax
import jax.numpy as jnp
from jax import lax
from jax.experimental import pallas as pl
from jax.experimental.pallas import tpu as pltpu
from jax.experimental.pallas import tpu_sc as plsc

N = 8192
APP_DIM = 48
KB = 1024
M = 64
GW = 64
SW = 16


def _sigmoid(x):
    return 1.0 / (1.0 + jnp.exp(-x))


def _diffuse_kernel(xyz_ref, vd_ref, n_ref, app_ref, noise_ref,
                    W1_ref, b1_ref, W2_ref, b2_ref,
                    diffuse_ref, r1_ref, table_ref, bg_ref):
    x3 = xyz_ref[...][:, 0:3]
    v = vd_ref[...]
    a = app_ref[...]
    W1 = W1_ref[...]
    h = (jnp.dot(x3, W1[0:3, :], preferred_element_type=jnp.float32)
         + jnp.dot(v, W1[3:6, :], preferred_element_type=jnp.float32)
         + jnp.dot(a, W1[6:54, :], preferred_element_type=jnp.float32)
         + b1_ref[...])
    h = jnp.maximum(h, 0.0)
    o = jnp.dot(h, W2_ref[...], preferred_element_type=jnp.float32) + b2_ref[...]
    diffuse = _sigmoid(o[:, 0:3])
    diffuse_ref[...] = diffuse
    r1 = _sigmoid(o[:, 6:7]) * 0.99 + 0.01
    r1_ref[...] = r1
    noise_app = a + 0.1 * noise_ref[...]
    blk = x3.shape[0]
    table_ref[...] = jnp.concatenate(
        [n_ref[...], v, x3, r1, noise_app, diffuse,
         jnp.zeros((blk, GW - 61), dtype=jnp.float32)], axis=1)
    bg_ref[...] = jnp.concatenate(
        [diffuse, jnp.zeros((blk, SW - 3), dtype=jnp.float32)], axis=1)


def _run_diffuse(xyzs, viewdirs, normals, app, noise, Wd1, bd1, Wd2, bd2):
    blk = 2048
    grid = (N // blk,)
    row = lambda i: (i, 0)
    full = lambda i: (0, 0)
    return pl.pallas_call(
        _diffuse_kernel,
        grid=grid,
        in_specs=[
            pl.BlockSpec((blk, 4), row),
            pl.BlockSpec((blk, 3), row),
            pl.BlockSpec((blk, 3), row),
            pl.BlockSpec((blk, APP_DIM), row),
            pl.BlockSpec((blk, APP_DIM), row),
            pl.BlockSpec((54, 64), full),
            pl.BlockSpec((1, 64), full),
            pl.BlockSpec((64, 8), full),
            pl.BlockSpec((1, 8), full),
        ],
        out_specs=[
            pl.BlockSpec((blk, 3), row),
            pl.BlockSpec((blk, 1), row),
            pl.BlockSpec((blk, GW), row),
            pl.BlockSpec((blk, SW), row),
        ],
        out_shape=[
            jax.ShapeDtypeStruct((N, 3), jnp.float32),
            jax.ShapeDtypeStruct((N, 1), jnp.float32),
            jax.ShapeDtypeStruct((N, GW), jnp.float32),
            jax.ShapeDtypeStruct((N, SW), jnp.float32),
        ],
    )(xyzs, viewdirs, normals, app, noise, Wd1, bd1, Wd2, bd2)


_NW = 32
_ROWS_PER_W = KB // _NW


@functools.lru_cache(maxsize=None)
def _sc_mesh():
    return plsc.VectorSubcoreMesh(core_axis_name="c", subcore_axis_name="s")


@functools.lru_cache(maxsize=None)
def _make_sc_gather():
    @functools.partial(
        pl.kernel,
        out_type=jax.ShapeDtypeStruct((KB, GW), jnp.float32),
        mesh=_sc_mesh(),
        compiler_params=pltpu.CompilerParams(use_tc_tiling_on_sc=False),
        scratch_types=[
            pltpu.VMEM((_ROWS_PER_W,), jnp.int32),
            pltpu.VMEM((_ROWS_PER_W, GW), jnp.float32),
            pltpu.SemaphoreType.DMA,
        ],
    )
    def _sc_gather(table_hbm, idx_hbm, out_hbm, idx_v, rows_v, sem):
        wid = lax.axis_index("s") * 2 + lax.axis_index("c")
        base = wid * _ROWS_PER_W
        pltpu.sync_copy(idx_hbm.at[pl.ds(base, _ROWS_PER_W)], idx_v)
        pltpu.async_copy(table_hbm.at[idx_v], rows_v, sem).wait()
        pltpu.sync_copy(rows_v, out_hbm.at[pl.ds(base, _ROWS_PER_W)])

    return _sc_gather


def _ray_kernel(gT_ref, u1T_ref, u2T_ref, maskT_ref, WrT_ref, b1T_ref,
                WcT_ref, W2T_ref, b2T_ref, Wenv_ref, outT_ref):
    gT = gT_ref[...]
    nx, ny, nz = gT[0:1, :], gT[1:2, :], gT[2:3, :]
    vx, vy, vz = -gT[3:4, :], -gT[4:5, :], -gT[5:6, :]
    ea = gT[9:10, :]

    cond = jnp.abs(nz) < 0.9
    tux = jnp.where(cond, -ny, 0.0)
    tuy = jnp.where(cond, nx, -nz)
    tuz = jnp.where(cond, 0.0, ny)
    tn = jnp.sqrt(tux * tux + tuy * tuy + tuz * tuz) + 1e-8
    tx, ty, tz = tux / tn, tuy / tn, tuz / tn
    bx = ny * tz - nz * ty
    by = nz * tx - nx * tz
    bz = nx * ty - ny * tx

    alpha = ea * ea
    a2 = alpha * alpha
    u1 = u1T_ref[...]
    u2 = u2T_ref[...]
    phi = (2.0 * jnp.pi) * u1
    ct = jnp.sqrt((1.0 - u2) / (1.0 + (a2 - 1.0) * u2))
    st = jnp.sqrt(jnp.clip(1.0 - ct * ct, 0.0, 1.0))
    h0 = st * jnp.cos(phi)
    h1 = st * jnp.sin(phi)
    h2 = ct

    hwx = tx * h0 + bx * h1 + nx * h2
    hwy = ty * h0 + by * h1 + ny * h2
    hwz = tz * h0 + bz * h1 + nz * h2
    hn = jnp.sqrt(hwx * hwx + hwy * hwy + hwz * hwz) + 1e-8
    hwx, hwy, hwz = hwx / hn, hwy / hn, hwz / hn

    dvh = vx * hwx + vy * hwy + vz * hwz
    lxu = 2.0 * dvh * hwx - vx
    lyu = 2.0 * dvh * hwy - vy
    lzu = 2.0 * dvh * hwz - vz
    ln = jnp.sqrt(lxu * lxu + lyu * lyu + lzu * lzu) + 1e-8
    lx, ly, lz = lxu / ln, lyu / ln, lzu / ln

    hxu = (vx + lx) * 0.5
    hyu = (vy + ly) * 0.5
    hzu = (vz + lz) * 0.5
    hn2 = jnp.sqrt(hxu * hxu + hyu * hyu + hzu * hzu) + 1e-8
    hx, hy, hz = hxu / hn2, hyu / hn2, hzu / hn2

    noh = jnp.clip(nx * hx + ny * hy + nz * hz, 1e-6, 1.0)
    voh = jnp.clip(jnp.abs(hx * vx + hy * vy + hz * vz), 1e-6, 1.0)
    mip = jnp.log(alpha / (4.0 * noh * voh + 1e-8) + 1e-8)

    wenv = Wenv_ref[...]
    il = []
    for c in range(3):
        acc = (lx * wenv[0:1, c:c + 1] + ly * wenv[1:2, c:c + 1]
               + lz * wenv[2:3, c:c + 1] + mip * wenv[3:4, c:c + 1])
        il.append(_sigmoid(acc))

    dv0 = tx * lx + ty * ly + tz * lz
    dv1 = bx * lx + by * ly + bz * lz
    dv2 = nx * lx + ny * ly + nz * lz
    hv0 = tx * hx + ty * hy + tz * hz
    hv1 = bx * hx + by * hy + bz * hz
    hv2 = nx * hx + ny * hy + nz * hz

    row_inT = jnp.concatenate(
        [-gT[3:6, :], gT[0:3, :], gT[10:58, :], gT[9:10, :]], axis=0)
    row_acc = (jnp.dot(WrT_ref[...], row_inT, preferred_element_type=jnp.float32)
               + b1T_ref[...])

    WcT = WcT_ref[...]
    W2T = W2T_ref[...]
    b2T = b2T_ref[...]
    maskT = maskT_ref[...]
    cnt = jnp.sum(maskT, axis=0, keepdims=True) + 1e-8

    NACC = 4
    tints = [jnp.zeros((3, KB), dtype=jnp.float32) for _ in range(NACC)]
    brdfs = [jnp.zeros((3, KB), dtype=jnp.float32) for _ in range(NACC)]
    for m in range(M):
        cf = jnp.concatenate(
            [lx[m:m + 1, :], ly[m:m + 1, :], lz[m:m + 1, :],
             hv0[m:m + 1, :], hv1[m:m + 1, :], hv2[m:m + 1, :],
             dv0[m:m + 1, :], dv1[m:m + 1, :], dv2[m:m + 1, :]], axis=0)
        bh = jnp.maximum(
            row_acc + jnp.dot(WcT, cf, preferred_element_type=jnp.float32), 0.0)
        bw = _sigmoid(jnp.dot(W2T, bh, preferred_element_type=jnp.float32) + b2T)
        ilm = jnp.concatenate(
            [il[0][m:m + 1, :], il[1][m:m + 1, :], il[2][m:m + 1, :]], axis=0)
        mk = maskT[m:m + 1, :]
        k = m % NACC
        tints[k] = tints[k] + mk * ilm * bw
        brdfs[k] = brdfs[k] + mk * bw
    tint_acc = (tints[0] + tints[1]) + (tints[2] + tints[3])
    brdf_acc = (brdfs[0] + brdfs[1]) + (brdfs[2] + brdfs[3])

    spec_rows = [jnp.sum(maskT * il[c], axis=0, keepdims=True) for c in range(3)]
    packed = jnp.concatenate(
        [tint_acc / cnt + gT[58:61, :],
         brdf_acc / cnt,
         jnp.concatenate(spec_rows, axis=0) / cnt,
         jnp.zeros((SW - 9, KB), dtype=jnp.float32)], axis=0)
    outT_ref[...] = packed.T


def _run_rays(gbT, u1T, u2T, maskT, WrT, b1T, WcT, W2T, b2T, Wenv):
    full = lambda: (0, 0)
    return pl.pallas_call(
        _ray_kernel,
        in_specs=[
            pl.BlockSpec((GW, KB), full),
            pl.BlockSpec((M, KB), full),
            pl.BlockSpec((M, KB), full),
            pl.BlockSpec((M, KB), full),
            pl.BlockSpec((64, 55), full),
            pl.BlockSpec((64, 1), full),
            pl.BlockSpec((64, 9), full),
            pl.BlockSpec((3, 64), full),
            pl.BlockSpec((3, 1), full),
            pl.BlockSpec((4, 3), full),
        ],
        out_specs=pl.BlockSpec((KB, SW), full),
        out_shape=jax.ShapeDtypeStruct((KB, SW), jnp.float32),
    )(gbT, u1T, u2T, maskT, WrT, b1T, WcT, W2T, b2T, Wenv)


_ZROWS = N // 16
_SROWS = KB // 16


@functools.lru_cache(maxsize=None)
def _make_sc_scatter():
    @functools.partial(
        pl.kernel,
        out_type=jax.ShapeDtypeStruct((N, SW), jnp.float32),
        mesh=_sc_mesh(),
        compiler_params=pltpu.CompilerParams(use_tc_tiling_on_sc=False),
        scratch_types=[
            pltpu.VMEM((_ZROWS, SW), jnp.float32),
            pltpu.VMEM((_SROWS,), jnp.int32),
            pltpu.VMEM((_SROWS, SW), jnp.float32),
            pltpu.SemaphoreType.DMA,
        ],
    )
    def _sc_scatter(vals_hbm, idx_hbm, bg_hbm, out_hbm, zbuf, idx_v, vals_v, sem):
        cid = lax.axis_index("c")
        sid = lax.axis_index("s")

        @pl.when(cid == 0)
        def _():
            s = pl.ds(sid * _ZROWS, _ZROWS)
            pltpu.sync_copy(bg_hbm.at[s], zbuf)
            pltpu.sync_copy(zbuf, out_hbm.at[s])
            plsc.subcore_barrier()
            pltpu.sync_copy(idx_hbm.at[pl.ds(sid * _SROWS, _SROWS)], idx_v)
            pltpu.sync_copy(vals_hbm.at[pl.ds(sid * _SROWS, _SROWS)], vals_v)
            pltpu.async_copy(vals_v, out_hbm.at[idx_v], sem).wait()

    return _sc_scatter


def kernel(xyzs, app_features, viewdirs, normals, weights, feat_noise, u1, u2,
           Wd1, bd1, Wd2, bd2, Wb1, bb1, Wb2, bb2, Wenv,
           app_mask, bounce_idx, ray_rows, ray_cols, ray_mask):
    diffuse, r1, table, bg = _run_diffuse(
        xyzs, viewdirs, normals, app_features, feat_noise,
        Wd1, bd1.reshape(1, 64), Wd2, bd2.reshape(1, 8))

    gb = _make_sc_gather()(table, bounce_idx)

    maskT = ray_mask.astype(jnp.float32).T
    WrT = jnp.concatenate([Wb1[0:3], Wb1[6:9], Wb1[15:63], Wb1[63:64]], axis=0).T
    WcT = Wb1[jnp.array([3, 4, 5, 9, 10, 11, 12, 13, 14])].T
    vals = _run_rays(gb.T, u1.T, u2.T, maskT, WrT, bb1.reshape(64, 1),
                     WcT, Wb2.T, bb2.reshape(3, 1), Wenv)

    scat = _make_sc_scatter()(vals, bounce_idx, bg)

    return (scat[:, 0:3], diffuse, r1, scat[:, 3:6], scat[:, 6:9])
